# baseline (speedup 1.0000x reference)
import functools
import math

import jax
import jax.numpy as jnp
from jax import lax
from jax.experimental import pallas as pl
from jax.experimental.pallas import tpu as pltpu
from jax.experimental.pallas import tpu_sc as plsc

_ALPHA = 5.0
_CHUNK = 128


def _neftune_body(scale, n_chunks, n_cores, chunk_rows,
                  ids_hbm, table_hbm, noise_hbm, out_hbm,
                  idx_v, rows_v, noise_v, gsem, nsem):
    wid = lax.axis_index("s") * n_cores + lax.axis_index("c")
    wbase = wid * (n_chunks * chunk_rows)
    d = rows_v.shape[1]

    def chunk(ci, carry):
        base = wbase + ci * chunk_rows
        pltpu.sync_copy(ids_hbm.at[pl.ds(base, chunk_rows)], idx_v)
        gat = pltpu.async_copy(table_hbm.at[idx_v], rows_v, gsem)
        noi = pltpu.async_copy(noise_hbm.at[pl.ds(base, chunk_rows)],
                               noise_v, nsem)
        gat.wait()
        noi.wait()

        def row(i, c2):
            for j in range(d // 16):
                sl = pl.ds(j * 16, 16)
                rows_v[i, sl] = rows_v[i, sl] + noise_v[i, sl] * scale
            return c2

        lax.fori_loop(0, chunk_rows, row, 0, unroll=4)
        pltpu.sync_copy(rows_v, out_hbm.at[pl.ds(base, chunk_rows)])
        return carry

    lax.fori_loop(0, n_chunks, chunk, 0)


def kernel(input_ids, table, noise):
    b, s = input_ids.shape
    _, d = table.shape
    n = b * s
    scale = _ALPHA / math.sqrt(s * d)

    ids = input_ids.reshape(n).astype(jnp.int32)
    noise2 = noise.reshape(n, d)

    info = plsc.get_sparse_core_info()
    n_workers = info.num_cores * info.num_subcores
    per_worker = n // n_workers
    n_chunks = per_worker // _CHUNK

    mesh = plsc.VectorSubcoreMesh(core_axis_name="c", subcore_axis_name="s")
    body = functools.partial(_neftune_body, scale, n_chunks, info.num_cores,
                             _CHUNK)
    run = pl.kernel(
        body,
        out_type=jax.ShapeDtypeStruct((n, d), jnp.float32),
        mesh=mesh,
        scratch_types=[
            pltpu.VMEM((_CHUNK,), jnp.int32),
            pltpu.VMEM((_CHUNK, d), jnp.float32),
            pltpu.VMEM((_CHUNK, d), jnp.float32),
            pltpu.SemaphoreType.DMA,
            pltpu.SemaphoreType.DMA,
        ],
        compiler_params=pltpu.CompilerParams(use_tc_tiling_on_sc=False),
    )
    out = run(ids, table, noise2)
    return out.reshape(b, s, d)


# speedup vs baseline: 1.4626x; 1.4626x over previous
import functools
import math

import jax
import jax.numpy as jnp
from jax import lax
from jax.experimental import pallas as pl
from jax.experimental.pallas import tpu as pltpu
from jax.experimental.pallas import tpu_sc as plsc

_ALPHA = 5.0
_CHUNK = 256
_GATHER = 128
_NBUF = 4


def _neftune_body(scale, n_chunks, n_cores,
                  ids_hbm, table_hbm, noise_hbm, out_hbm, *scratch):
    idx = scratch[0:_NBUF]
    acc = scratch[_NBUF:2 * _NBUF]
    isem = scratch[2 * _NBUF:3 * _NBUF]
    nsem = scratch[3 * _NBUF:4 * _NBUF]
    gsem = scratch[4 * _NBUF:5 * _NBUF]
    osem = scratch[5 * _NBUF:6 * _NBUF]
    d = acc[0].shape[1]
    ng = _CHUNK // _GATHER

    wid = lax.axis_index("s") * n_cores + lax.axis_index("c")
    wbase = wid * (n_chunks * _CHUNK)

    def issue_inputs(i, b):
        base = wbase + i * _CHUNK
        pltpu.async_copy(ids_hbm.at[pl.ds(base, _CHUNK)], idx[b], isem[b])
        pltpu.async_copy(noise_hbm.at[pl.ds(base, _CHUNK)], acc[b], nsem[b])

    def wait_inputs(i, b):
        base = wbase + i * _CHUNK
        pltpu.make_async_copy(ids_hbm.at[pl.ds(base, _CHUNK)], idx[b],
                              isem[b]).wait()
        pltpu.make_async_copy(noise_hbm.at[pl.ds(base, _CHUNK)], acc[b],
                              nsem[b]).wait()

    def scale_acc(b):
        def row(r, c2):
            for j in range(d // 16):
                sl = pl.ds(j * 16, 16)
                acc[b][r, sl] = acc[b][r, sl] * scale
            return c2
        lax.fori_loop(0, _CHUNK, row, 0, unroll=4)

    def issue_gathers(b):
        for g in range(ng):
            gsl = pl.ds(g * _GATHER, _GATHER)
            pltpu.async_copy(table_hbm.at[idx[b].at[gsl]], acc[b].at[gsl],
                             gsem[b], add=True)

    def wait_gathers(b):
        for g in range(ng):
            gsl = pl.ds(g * _GATHER, _GATHER)
            pltpu.make_async_copy(table_hbm.at[idx[b].at[gsl]],
                                  acc[b].at[gsl], gsem[b]).wait()

    def issue_out(i, b):
        pltpu.async_copy(acc[b], out_hbm.at[pl.ds(wbase + i * _CHUNK, _CHUNK)],
                         osem[b])

    def wait_out(i, b):
        pltpu.make_async_copy(acc[b],
                              out_hbm.at[pl.ds(wbase + i * _CHUNK, _CHUNK)],
                              osem[b]).wait()

    issue_inputs(0, 0)
    issue_inputs(1, 1)

    def outer(io, carry):
        for b in range(_NBUF):
            i = io * _NBUF + b
            bm1 = (b - 1) % _NBUF
            bm2 = (b - 2) % _NBUF

            wait_inputs(i, b)
            scale_acc(b)
            issue_gathers(b)

            @pl.when(i >= 1)
            def _():
                wait_gathers(bm1)
                issue_out(i - 1, bm1)

            @pl.when(i >= 2)
            def _():
                wait_out(i - 2, bm2)

            @pl.when(i + 2 < n_chunks)
            def _():
                issue_inputs(i + 2, bm2)
        return carry

    lax.fori_loop(0, n_chunks // _NBUF, outer, 0)

    b_last = (n_chunks - 1) % _NBUF
    b_prev = (n_chunks - 2) % _NBUF
    wait_gathers(b_last)
    issue_out(n_chunks - 1, b_last)
    wait_out(n_chunks - 2, b_prev)
    wait_out(n_chunks - 1, b_last)


def kernel(input_ids, table, noise):
    b, s = input_ids.shape
    _, d = table.shape
    n = b * s
    scale = _ALPHA / math.sqrt(s * d)

    ids = input_ids.reshape(n).astype(jnp.int32)
    noise2 = noise.reshape(n, d)

    info = plsc.get_sparse_core_info()
    n_workers = info.num_cores * info.num_subcores
    per_worker = n // n_workers
    n_chunks = per_worker // _CHUNK

    mesh = plsc.VectorSubcoreMesh(core_axis_name="c", subcore_axis_name="s")
    body = functools.partial(_neftune_body, scale, n_chunks, info.num_cores)
    scratch = ([pltpu.VMEM((_CHUNK,), jnp.int32) for _ in range(_NBUF)]
               + [pltpu.VMEM((_CHUNK, d), jnp.float32) for _ in range(_NBUF)]
               + [pltpu.SemaphoreType.DMA for _ in range(4 * _NBUF)])
    run = pl.kernel(
        body,
        out_type=jax.ShapeDtypeStruct((n, d), jnp.float32),
        mesh=mesh,
        scratch_types=scratch,
        compiler_params=pltpu.CompilerParams(use_tc_tiling_on_sc=False),
    )
    out = run(ids, table, noise2)
    return out.reshape(b, s, d)
